# baseline (speedup 1.0000x reference)
import functools

import jax
import jax.numpy as jnp
from jax import lax
from jax.experimental import pallas as pl
from jax.experimental.pallas import tpu as pltpu
from jax.experimental.pallas import tpu_sc as plsc

TAG = 32
_SIGN = -(2 ** 31)

_NC = 2
_NS = 16
_NW = _NC * _NS

_CH = 16


def _tc_body(x_ref, wt_ref, b_ref, kc_ref, t_ref):
    x = x_ref[...]
    xc = x - jnp.mean(x, axis=1, keepdims=True)
    kc = jnp.dot(xc, wt_ref[...], preferred_element_type=jnp.float32)
    kc = kc + b_ref[...]

    b = lax.bitcast_convert_type(kc, jnp.int32)
    s = jnp.where(b >= 0, b, b ^ jnp.int32(0x7FFFFFFF))

    rows = kc.shape[0]

    def step(i, cur):
        bit = lax.shift_left(jnp.int32(1), jnp.int32(31) - i)
        cand_u = cur | bit
        cand_s = cand_u ^ jnp.int32(_SIGN)
        cnt = jnp.sum((s >= cand_s).astype(jnp.int32), axis=1, keepdims=True)
        return jnp.where(cnt >= TAG, cand_u, cur)

    cur = lax.fori_loop(0, 32, step, jnp.zeros((rows, 1), jnp.int32))
    t_s = cur ^ jnp.int32(_SIGN)
    t_b = jnp.where(t_s >= 0, t_s, t_s ^ jnp.int32(0x7FFFFFFF))
    kc_ref[...] = kc
    t_ref[...] = jnp.broadcast_to(
        lax.bitcast_convert_type(t_b, jnp.float32), t_ref.shape)


def _project_and_threshold(x, wt, b2):
    batch, in_dim = x.shape
    out_dim = wt.shape[1]
    br = min(512, batch)
    return pl.pallas_call(
        _tc_body,
        grid=(batch // br,),
        in_specs=[
            pl.BlockSpec((br, in_dim), lambda i: (i, 0)),
            pl.BlockSpec((in_dim, out_dim), lambda i: (0, 0)),
            pl.BlockSpec((1, out_dim), lambda i: (0, 0)),
        ],
        out_specs=[
            pl.BlockSpec((br, out_dim), lambda i: (i, 0)),
            pl.BlockSpec((br, 16), lambda i: (i, 0)),
        ],
        out_shape=[
            jax.ShapeDtypeStruct((batch, out_dim), jnp.float32),
            jax.ShapeDtypeStruct((batch, 16), jnp.float32),
        ],
        compiler_params=pltpu.CompilerParams(
            dimension_semantics=("parallel",)
        ),
    )(x, wt, b2)


def _sc_mask(kc_flat, t_flat, batch, out_dim):
    rows_per_w = batch // _NW
    chunks = rows_per_w // _CH
    mesh = plsc.VectorSubcoreMesh(
        core_axis_name="c", subcore_axis_name="s",
        num_cores=_NC, num_subcores=_NS,
    )

    @functools.partial(
        pl.kernel,
        out_type=jax.ShapeDtypeStruct((batch * out_dim,), jnp.float32),
        mesh=mesh,
        scratch_types=[
            pltpu.VMEM((_CH * out_dim,), jnp.float32),
            pltpu.VMEM((rows_per_w * 16,), jnp.float32),
        ],
    )
    def k(kc_hbm, t_hbm, out_hbm, buf, t_v):
        wid = lax.axis_index("s") * _NC + lax.axis_index("c")
        row0 = wid * rows_per_w
        pltpu.sync_copy(t_hbm.at[pl.ds(row0 * 16, rows_per_w * 16)], t_v)

        def chunk_body(c, _):
            base = (row0 + c * _CH) * out_dim
            pltpu.sync_copy(kc_hbm.at[pl.ds(base, _CH * out_dim)], buf)
            for r in range(_CH):
                tvec = t_v[pl.ds((c * _CH + r) * 16, 16)]

                def slice_body(i, _, r=r, tvec=tvec):
                    for u in range(8):
                        off = r * out_dim + i * 128 + u * 16
                        v = buf[pl.ds(off, 16)]
                        buf[pl.ds(off, 16)] = jnp.where(
                            v >= tvec, v, jnp.zeros((16,), jnp.float32))
                    return 0

                lax.fori_loop(0, out_dim // 128, slice_body, 0)
            pltpu.sync_copy(buf, out_hbm.at[pl.ds(base, _CH * out_dim)])
            return 0

        lax.fori_loop(0, chunks, chunk_body, 0)

    return k(kc_flat, t_flat)


@jax.jit
def kernel(x, W, b):
    batch, in_dim = x.shape
    out_dim = W.shape[0]
    wt = W.T
    b2 = b.reshape(1, out_dim)
    kc, t = _project_and_threshold(x, wt, b2)
    out_flat = _sc_mask(kc.reshape(-1), t.reshape(-1), batch, out_dim)
    return out_flat.reshape(batch, out_dim)


# speedup vs baseline: 1.2697x; 1.2697x over previous
import functools

import jax
import jax.numpy as jnp
from jax import lax
from jax.experimental import pallas as pl
from jax.experimental.pallas import tpu as pltpu
from jax.experimental.pallas import tpu_sc as plsc

TAG = 32
_SIGN = -(2 ** 31)

_NC = 2
_NS = 16
_NW = _NC * _NS

_CH = 16


def _tc_body(x_ref, wt_ref, b_ref, kc_ref, t_ref):
    x = x_ref[...]
    xc = x - jnp.mean(x, axis=1, keepdims=True)
    kc = jnp.dot(xc, wt_ref[...], preferred_element_type=jnp.float32)
    kc = kc + b_ref[...]

    b = lax.bitcast_convert_type(kc, jnp.int32)
    s = jnp.where(b >= 0, b, b ^ jnp.int32(0x7FFFFFFF))

    rows = kc.shape[0]

    def step(i, cur):
        bit = lax.shift_left(jnp.int32(1), jnp.int32(31) - i)
        cand_u = cur | bit
        cand_s = cand_u ^ jnp.int32(_SIGN)
        cnt = jnp.sum((s >= cand_s).astype(jnp.int32), axis=1, keepdims=True)
        return jnp.where(cnt >= TAG, cand_u, cur)

    cur = lax.fori_loop(0, 32, step, jnp.zeros((rows, 1), jnp.int32))
    t_s = cur ^ jnp.int32(_SIGN)
    t_b = jnp.where(t_s >= 0, t_s, t_s ^ jnp.int32(0x7FFFFFFF))
    kc_ref[...] = kc
    t_ref[...] = jnp.broadcast_to(
        lax.bitcast_convert_type(t_b, jnp.float32), t_ref.shape)


def _project_and_threshold(x, wt, b2):
    batch, in_dim = x.shape
    out_dim = wt.shape[1]
    br = min(512, batch)
    return pl.pallas_call(
        _tc_body,
        grid=(batch // br,),
        in_specs=[
            pl.BlockSpec((br, in_dim), lambda i: (i, 0)),
            pl.BlockSpec((in_dim, out_dim), lambda i: (0, 0)),
            pl.BlockSpec((1, out_dim), lambda i: (0, 0)),
        ],
        out_specs=[
            pl.BlockSpec((br, out_dim), lambda i: (i, 0)),
            pl.BlockSpec((br, 16), lambda i: (i, 0)),
        ],
        out_shape=[
            jax.ShapeDtypeStruct((batch, out_dim), jnp.float32),
            jax.ShapeDtypeStruct((batch, 16), jnp.float32),
        ],
        compiler_params=pltpu.CompilerParams(
            dimension_semantics=("parallel",)
        ),
    )(x, wt, b2)


def _sc_mask(kc, t, batch, out_dim):
    rows_per_w = batch // _NW
    chunks = rows_per_w // _CH
    mesh = plsc.VectorSubcoreMesh(
        core_axis_name="c", subcore_axis_name="s",
        num_cores=_NC, num_subcores=_NS,
    )

    @functools.partial(
        pl.kernel,
        out_type=jax.ShapeDtypeStruct((batch, out_dim), jnp.float32),
        mesh=mesh,
        scratch_types=[
            pltpu.VMEM((_CH, out_dim), jnp.float32),
            pltpu.VMEM((rows_per_w, 16), jnp.float32),
        ],
    )
    def k(kc_hbm, t_hbm, out_hbm, buf, t_v):
        wid = lax.axis_index("s") * _NC + lax.axis_index("c")
        row0 = wid * rows_per_w
        pltpu.sync_copy(t_hbm.at[pl.ds(row0, rows_per_w)], t_v)

        def chunk_body(c, _):
            base = row0 + c * _CH
            pltpu.sync_copy(kc_hbm.at[pl.ds(base, _CH)], buf)
            for r in range(_CH):
                tvec = t_v[c * _CH + r, :]

                def slice_body(i, _, r=r, tvec=tvec):
                    for u in range(8):
                        off = i * 128 + u * 16
                        v = buf[r, pl.ds(off, 16)]
                        buf[r, pl.ds(off, 16)] = jnp.where(
                            v >= tvec, v, jnp.zeros((16,), jnp.float32))
                    return 0

                lax.fori_loop(0, out_dim // 128, slice_body, 0)
            pltpu.sync_copy(buf, out_hbm.at[pl.ds(base, _CH)])
            return 0

        lax.fori_loop(0, chunks, chunk_body, 0)

    return k(kc, t)


@jax.jit
def kernel(x, W, b):
    batch, in_dim = x.shape
    out_dim = W.shape[0]
    wt = W.T
    b2 = b.reshape(1, out_dim)
    kc, t = _project_and_threshold(x, wt, b2)
    return _sc_mask(kc, t, batch, out_dim)
